# baseline (speedup 1.0000x reference)
import functools

import jax
import jax.numpy as jnp
from jax import lax
from jax.experimental import pallas as pl
from jax.experimental.pallas import tpu as pltpu
from jax.experimental.pallas import tpu_sc as plsc

_VOCAB = 1000000
_EMBED = 64
_BATCH = 16384
_HIST = 200

_BB = _BATCH // 128
_HG = _HIST // 8
_ITEMS = _HIST * _BB
_NBUF = 2


def _make_gather():
    info = plsc.get_sparse_core_info()
    nw = info.num_cores * info.num_subcores
    per_w = _ITEMS // nw
    assert per_w % _NBUF == 0

    mesh = plsc.VectorSubcoreMesh(core_axis_name="c", subcore_axis_name="s")

    @functools.partial(
        pl.kernel,
        out_type=jax.ShapeDtypeStruct((_HIST, 8, _BB, 8, 128), jnp.float32),
        mesh=mesh,
        scratch_types=[
            pltpu.VMEM((_NBUF, 128), jnp.int32),
            pltpu.VMEM((_NBUF, 128, _EMBED), jnp.float32),
            pltpu.VMEM((_NBUF, 8, 8, 128), jnp.float32),
            [pltpu.SemaphoreType.DMA] * _NBUF,
            [pltpu.SemaphoreType.DMA] * _NBUF,
            [pltpu.SemaphoreType.DMA] * _NBUF,
        ],
        compiler_params=pltpu.CompilerParams(
            use_tc_tiling_on_sc=False, needs_layout_passes=False
        ),
    )
    def gather(table_hbm, idx_hbm, out_hbm, idx_v, rows_v, trans_v,
               gsem, ssem, isem):
        wid = lax.axis_index("s") * info.num_cores + lax.axis_index("c")
        t_base = wid * per_w
        iota16 = lax.iota(jnp.int32, 16)

        def coords(t_loc):
            t = t_base + jnp.minimum(t_loc, per_w - 1)
            h = t // _BB
            bb = t % _BB
            return h, bb

        def idx_copy(t_loc, b, sem):
            h, bb = coords(t_loc)
            return pltpu.make_async_copy(
                idx_hbm.at[h // 8, bb, h % 8],
                idx_v.at[b],
                sem,
            )

        def gather_copy(b, sem):
            return pltpu.make_async_copy(
                table_hbm.at[idx_v.at[b]],
                rows_v.at[b],
                sem,
            )

        def store_copies(t_loc, b, sem):
            h, bb = coords(t_loc)
            return [
                pltpu.make_async_copy(
                    trans_v.at[b, eg],
                    out_hbm.at[h, eg, bb],
                    sem,
                )
                for eg in range(8)
            ]

        def store_drain(t_loc, b, sem):
            for cp in store_copies(t_loc, b, sem):
                cp.wait()

        def transpose(b):
            def tbody(k, carry):
                rowidx = k * 16 + iota16
                for e in range(_EMBED):
                    vals = plsc.load_gather(
                        rows_v.at[b], [rowidx, jnp.full((16,), e, jnp.int32)]
                    )
                    trans_v[b, e // 8, e % 8, pl.ds(k * 16, 16)] = vals
                return carry

            lax.fori_loop(0, 8, tbody, 0)

        idx_copy(0, 0, isem[0]).start()

        def body(i, carry):
            for b in range(_NBUF):
                t_loc = i * _NBUF + b
                bp = 1 - b
                idx_copy(t_loc, b, isem[b]).wait()
                if b == 0:
                    @pl.when(i > 0)
                    def _():
                        gather_copy(bp, gsem[bp]).wait()
                else:
                    gather_copy(bp, gsem[bp]).wait()
                gather_copy(b, gsem[b]).start()
                idx_copy(t_loc + 1, bp, isem[bp]).start()
                def retire_prev():
                    store_drain(t_loc - 1, bp, ssem[bp])
                    transpose(bp)
                    for cp in store_copies(t_loc - 1, bp, ssem[bp]):
                        cp.start()

                def retire_prev_nodrain():
                    transpose(bp)
                    for cp in store_copies(t_loc - 1, bp, ssem[bp]):
                        cp.start()

                if b == 0:
                    @pl.when(i > 1)
                    def _():
                        retire_prev()
                    @pl.when(i == 1)
                    def _():
                        retire_prev_nodrain()
                else:
                    @pl.when(i > 0)
                    def _():
                        retire_prev()
                    @pl.when(i == 0)
                    def _():
                        retire_prev_nodrain()
            return carry

        lax.fori_loop(0, per_w // _NBUF, body, 0)

        last = per_w - 1
        bl = last % _NBUF
        gather_copy(bl, gsem[bl]).wait()
        store_drain(last, bl, ssem[bl])
        transpose(bl)
        for cp in store_copies(last, bl, ssem[bl]):
            cp.start()
        store_drain(last - 1, 1 - bl, ssem[1 - bl])
        store_drain(last, bl, ssem[bl])
        idx_copy(per_w, 1 - bl, isem[1 - bl]).wait()

    return gather


def kernel(input_ids, table):
    idx4 = (
        input_ids.astype(jnp.int32)
        .reshape(_BB, 128, _HG, 8)
        .transpose(2, 0, 3, 1)
    )
    out5 = _make_gather()(table, idx4)
    return out5.transpose(2, 4, 0, 1, 3).reshape(_BATCH, _HIST, _EMBED)
